# baseline (speedup 1.0000x reference)
import functools

import jax
import jax.numpy as jnp
from jax import lax
from jax.experimental import pallas as pl
from jax.experimental.pallas import tpu as pltpu
from jax.experimental.pallas import tpu_sc as plsc

NUM_CODES = 1024
CODE_DIM = 64
ROWS_PER_BLOCK = 512
N_CHUNKS = 4


def _codes_body(x_ref, cbm2_ref, cn_ref, out_ref):
    x = x_ref[...]
    cbm2 = cbm2_ref[...]
    m2 = lax.dot_general(x, cbm2, (((1,), (1,)), ((), ())),
                         preferred_element_type=jnp.float32)
    rn = jnp.sum(x * x, axis=1, keepdims=True)
    d2 = (rn + m2) + cn_ref[...]
    out_ref[0, 0, :] = jnp.argmin(d2, axis=1).astype(jnp.int32)


def _compute_codes_chunk(flat, cbm2, cn, chunk, nblk_chunk):
    base = chunk * nblk_chunk
    codes3 = pl.pallas_call(
        _codes_body,
        grid=(nblk_chunk,),
        in_specs=[
            pl.BlockSpec((ROWS_PER_BLOCK, CODE_DIM),
                         lambda i, b=base: (b + i, 0)),
            pl.BlockSpec((NUM_CODES, CODE_DIM), lambda i: (0, 0)),
            pl.BlockSpec((1, NUM_CODES), lambda i: (0, 0)),
        ],
        out_specs=pl.BlockSpec((1, 1, ROWS_PER_BLOCK), lambda i: (i, 0, 0)),
        out_shape=jax.ShapeDtypeStruct((nblk_chunk, 1, ROWS_PER_BLOCK),
                                       jnp.int32),
    )(flat, cbm2, cn)
    return codes3.reshape(nblk_chunk * ROWS_PER_BLOCK)


def _make_sc_gather(n_rows):
    info = plsc.get_sparse_core_info()
    nw = info.num_cores * info.num_subcores
    b_per_w = n_rows // nw
    mesh = plsc.VectorSubcoreMesh(core_axis_name="c", subcore_axis_name="s")

    @functools.partial(
        pl.kernel,
        mesh=mesh,
        out_type=jax.ShapeDtypeStruct((n_rows, CODE_DIM), jnp.float32),
        scratch_types=[
            pltpu.VMEM((b_per_w,), jnp.int32),
            pltpu.VMEM((b_per_w, CODE_DIM), jnp.float32),
            pltpu.SemaphoreType.DMA,
        ],
        compiler_params=pltpu.CompilerParams(use_tc_tiling_on_sc=False),
    )
    def gather(table_hbm, idx_hbm, out_hbm, idx_v, rows_v, sem):
        wid = lax.axis_index("s") * info.num_cores + lax.axis_index("c")
        base = wid * b_per_w
        pltpu.sync_copy(idx_hbm.at[pl.ds(base, b_per_w)], idx_v)
        pltpu.async_copy(table_hbm.at[idx_v], rows_v, sem).wait()
        pltpu.sync_copy(rows_v, out_hbm.at[pl.ds(base, b_per_w)])

    return gather


def kernel(inputs, codebook):
    b, s, d = inputs.shape
    n = b * s
    flat = inputs.reshape(n, d)
    cbm2 = -2.0 * codebook
    cn = jnp.sum(codebook * codebook, axis=1)[None, :]
    rows_chunk = n // N_CHUNKS
    nblk_chunk = rows_chunk // ROWS_PER_BLOCK
    sc_gather = _make_sc_gather(rows_chunk)
    codes_parts = []
    quant_parts = []
    for c in range(N_CHUNKS):
        codes_c = _compute_codes_chunk(flat, cbm2, cn, c, nblk_chunk)
        quant_parts.append(sc_gather(codebook, codes_c))
        codes_parts.append(codes_c)
    codes_flat = jnp.concatenate(codes_parts)
    quantized = jnp.concatenate(quant_parts, axis=0)
    return quantized.reshape(inputs.shape), codes_flat.reshape(b, s)


# speedup vs baseline: 1.1148x; 1.1148x over previous
import functools

import jax
import jax.numpy as jnp
from jax import lax
from jax.experimental import pallas as pl
from jax.experimental.pallas import tpu as pltpu
from jax.experimental.pallas import tpu_sc as plsc

NUM_CODES = 1024
CODE_DIM = 64
ROWS_PER_BLOCK = 512
N_CHUNKS = 1


def _codes_body(x_ref, cbm2_ref, cn_ref, out_ref):
    x = x_ref[...]
    cbm2 = cbm2_ref[...]
    m2 = lax.dot_general(x, cbm2, (((1,), (1,)), ((), ())),
                         preferred_element_type=jnp.float32)
    rn = jnp.sum(x * x, axis=1, keepdims=True)
    d2 = (rn + m2) + cn_ref[...]
    out_ref[0, 0, :] = jnp.argmin(d2, axis=1).astype(jnp.int32)


def _compute_codes_chunk(flat, cbm2, cn, chunk, nblk_chunk):
    base = chunk * nblk_chunk
    codes3 = pl.pallas_call(
        _codes_body,
        grid=(nblk_chunk,),
        in_specs=[
            pl.BlockSpec((ROWS_PER_BLOCK, CODE_DIM),
                         lambda i, b=base: (b + i, 0)),
            pl.BlockSpec((NUM_CODES, CODE_DIM), lambda i: (0, 0)),
            pl.BlockSpec((1, NUM_CODES), lambda i: (0, 0)),
        ],
        out_specs=pl.BlockSpec((1, 1, ROWS_PER_BLOCK), lambda i: (i, 0, 0)),
        out_shape=jax.ShapeDtypeStruct((nblk_chunk, 1, ROWS_PER_BLOCK),
                                       jnp.int32),
    )(flat, cbm2, cn)
    return codes3.reshape(nblk_chunk * ROWS_PER_BLOCK)


def _make_sc_gather(n_rows):
    info = plsc.get_sparse_core_info()
    nw = info.num_cores * info.num_subcores
    b_per_w = n_rows // nw
    mesh = plsc.VectorSubcoreMesh(core_axis_name="c", subcore_axis_name="s")

    @functools.partial(
        pl.kernel,
        mesh=mesh,
        out_type=jax.ShapeDtypeStruct((n_rows, CODE_DIM), jnp.float32),
        scratch_types=[
            pltpu.VMEM((b_per_w,), jnp.int32),
            pltpu.VMEM((b_per_w, CODE_DIM), jnp.float32),
            pltpu.SemaphoreType.DMA,
        ],
        compiler_params=pltpu.CompilerParams(use_tc_tiling_on_sc=False),
    )
    def gather(table_hbm, idx_hbm, out_hbm, idx_v, rows_v, sem):
        wid = lax.axis_index("s") * info.num_cores + lax.axis_index("c")
        base = wid * b_per_w
        pltpu.sync_copy(idx_hbm.at[pl.ds(base, b_per_w)], idx_v)
        pltpu.async_copy(table_hbm.at[idx_v], rows_v, sem).wait()
        pltpu.sync_copy(rows_v, out_hbm.at[pl.ds(base, b_per_w)])

    return gather


def kernel(inputs, codebook):
    b, s, d = inputs.shape
    n = b * s
    flat = inputs.reshape(n, d)
    cbm2 = -2.0 * codebook
    cn = jnp.sum(codebook * codebook, axis=1)[None, :]
    rows_chunk = n // N_CHUNKS
    nblk_chunk = rows_chunk // ROWS_PER_BLOCK
    sc_gather = _make_sc_gather(rows_chunk)
    codes_parts = []
    quant_parts = []
    for c in range(N_CHUNKS):
        codes_c = _compute_codes_chunk(flat, cbm2, cn, c, nblk_chunk)
        quant_parts.append(sc_gather(codebook, codes_c))
        codes_parts.append(codes_c)
    codes_flat = jnp.concatenate(codes_parts)
    quantized = jnp.concatenate(quant_parts, axis=0)
    return quantized.reshape(inputs.shape), codes_flat.reshape(b, s)
